# Initial kernel scaffold; baseline (speedup 1.0000x reference)
import jax
import jax.numpy as jnp
from jax.experimental import pallas as pl


def kernel(obs, action, w1o, w1a, b1, rx, rte, ewa, eba, ewb, ebb, rexp, w6pack, b6pack):
    raise NotImplementedError("write your pallas kernel here")


# speedup vs baseline: 1.8742x; 1.8742x over previous
import jax
import jax.numpy as jnp
from jax.experimental import pallas as pl
from jax.experimental.pallas import tpu as pltpu

MU = 0.01
LANES = 128


def _qnet_kernel(obs_ref, act_ref, w1o_ref, w1a_ref, b1_ref, rx_ref, rte_ref,
                 ewa_ref, eba_ref, ewb_ref, ebb_ref, w6_ref, b6_ref, out_ref):
    f32 = jnp.float32
    T = rte_ref.shape[0]
    H2 = w1o_ref.shape[1]
    H = H2 // 2
    E2 = rx_ref.shape[1]
    E = E2 // 2
    EH = E * H
    OB = obs_ref.shape[1] - T

    base = obs_ref[:, :OB]
    onehot = obs_ref[:, OB:]

    x1 = (jnp.dot(base, w1o_ref[...], preferred_element_type=f32)
          + jnp.dot(act_ref[...], w1a_ref[...], preferred_element_type=f32)
          + b1_ref[...])
    x1 = jnp.maximum(x1, 0.0)

    logits = (jnp.dot(x1, rx_ref[...], preferred_element_type=f32)
              + jnp.dot(onehot, rte_ref[...], preferred_element_type=f32))
    grp = jax.lax.broadcasted_iota(jnp.int32, logits.shape, 1) >= E
    neg = jnp.float32(-jnp.inf)
    m1 = jnp.max(jnp.where(grp, neg, logits), axis=-1, keepdims=True)
    m2 = jnp.max(jnp.where(grp, logits, neg), axis=-1, keepdims=True)
    e = jnp.exp(logits - jnp.where(grp, m2, m1))
    s1 = jnp.sum(jnp.where(grp, 0.0, e), axis=-1, keepdims=True)
    s2 = jnp.sum(jnp.where(grp, e, 0.0), axis=-1, keepdims=True)
    ew = e / jnp.where(grp, s2, s1)

    h1 = jnp.maximum(
        jnp.dot(x1[:, :H], ewa_ref[:H, :EH], preferred_element_type=f32)
        + eba_ref[:, :EH], 0.0)
    h2 = jnp.maximum(
        jnp.dot(x1[:, H:], ewa_ref[H:, EH:], preferred_element_type=f32)
        + eba_ref[:, EH:], 0.0)

    npair = E // 2
    gmix1 = jnp.zeros((x1.shape[0], H), f32)
    gmix2 = jnp.zeros((x1.shape[0], H), f32)
    for p in range(npair):
        o = 2 * H * p
        g = jnp.maximum(
            jnp.dot(h1[:, o:o + 2 * H],
                    ewb_ref[o:o + 2 * H, o:o + 2 * H],
                    preferred_element_type=f32)
            + ebb_ref[:, o:o + 2 * H], 0.0)
        gmix1 = (gmix1 + ew[:, 2 * p:2 * p + 1] * g[:, :H]
                 + ew[:, 2 * p + 1:2 * p + 2] * g[:, H:])
    for p in range(npair):
        o = 2 * H * p
        g = jnp.maximum(
            jnp.dot(h2[:, o:o + 2 * H],
                    ewb_ref[EH + o:EH + o + 2 * H, EH + o:EH + o + 2 * H],
                    preferred_element_type=f32)
            + ebb_ref[:, EH + o:EH + o + 2 * H], 0.0)
        gmix2 = (gmix2 + ew[:, E + 2 * p:E + 2 * p + 1] * g[:, :H]
                 + ew[:, E + 2 * p + 1:E + 2 * p + 2] * g[:, H:])

    w6s = jnp.concatenate([w6_ref[:H, :], w6_ref[EH:EH + H, :]], axis=0)
    q12 = (jnp.dot(jnp.concatenate([gmix1, gmix2], axis=1), w6s,
                   preferred_element_type=f32)
           + b6_ref[...])

    reg = (-(1.0 / E) * MU
           * jnp.sum(jnp.log(ew + 1e-6), axis=-1, keepdims=True))

    col = jax.lax.broadcasted_iota(jnp.int32, out_ref.shape, 1)
    out_ref[...] = jnp.where(col == 2, reg, q12)


def _pick_tile(B, cap=512):
    if B <= cap:
        return B
    for tb in range(cap, 7, -8):
        if B % tb == 0:
            return tb
    return B


def kernel(obs, action, w1o, w1a, b1, rx, rte, ewa, eba, ewb, ebb,
           rexp, w6pack, b6pack):
    B = obs.shape[0]
    OBT = obs.shape[1]
    A = action.shape[1]
    T = rte.shape[0]
    H2 = w1o.shape[1]
    E2 = rx.shape[1]
    EH2 = ewa.shape[1]

    TB = _pick_tile(B)
    grid = (B // TB,)
    row = lambda i: (i, 0)
    rep = lambda i: (0, 0)

    H = H2 // 2
    flops = 2 * B * (OBT * H2 + A * H2 + H2 * E2 + T * E2
                     + H * EH2 + 2 * H * EH2 + H2 * LANES)
    bytes_accessed = 4 * (B * (OBT + A + LANES)
                          + OBT * H2 + A * H2 + H2 + H2 * E2 + T * E2
                          + H2 * EH2 + EH2 + EH2 * EH2 + EH2
                          + EH2 * LANES + LANES)

    out = pl.pallas_call(
        _qnet_kernel,
        out_shape=jax.ShapeDtypeStruct((B, LANES), jnp.float32),
        grid=grid,
        in_specs=[
            pl.BlockSpec((TB, OBT), row),
            pl.BlockSpec((TB, A), row),
            pl.BlockSpec((OBT - T, H2), rep),
            pl.BlockSpec((A, H2), rep),
            pl.BlockSpec((1, H2), rep),
            pl.BlockSpec((H2, E2), rep),
            pl.BlockSpec((T, E2), rep),
            pl.BlockSpec((H2, EH2), rep),
            pl.BlockSpec((1, EH2), rep),
            pl.BlockSpec((EH2, EH2), rep),
            pl.BlockSpec((1, EH2), rep),
            pl.BlockSpec((EH2, LANES), rep),
            pl.BlockSpec((1, LANES), rep),
        ],
        out_specs=pl.BlockSpec((TB, LANES), row),
        compiler_params=pltpu.CompilerParams(
            dimension_semantics=("parallel",)),
        cost_estimate=pl.CostEstimate(
            flops=flops, transcendentals=B * (2 * E2 + 2),
            bytes_accessed=bytes_accessed),
    )(obs, action, w1o, w1a, b1, rx, rte, ewa, eba, ewb, ebb,
      w6pack, b6pack)

    return out[:, 0:1], out[:, 1:2], out[:, 2]
